# baseline (speedup 1.0000x reference)
import jax
import jax.numpy as jnp
from jax.experimental import pallas as pl
from jax.experimental.pallas import tpu as pltpu


def _pos_embed_kernel(emb_ref, out_ref, t_ref, sems):
    i = pl.program_id(0)
    n = pl.num_programs(0)
    B, F, L = out_ref.shape
    LT = L // n

    def copies(step):
        return [
            pltpu.make_async_copy(
                t_ref.at[step],
                out_ref.at[b, :, pl.ds(step * LT, LT)],
                sems.at[step, b],
            )
            for b in range(B)
        ]

    t_ref[i] = emb_ref[...].T
    for cp in copies(i):
        cp.start()

    @pl.when(i == n - 1)
    def _():
        for s in range(t_ref.shape[0]):
            for cp in copies(s):
                cp.wait()


def kernel(x, mask, row_embed):
    B = x.shape[0]
    F = x.shape[1]
    L = x.shape[-1]
    LT = 256
    return pl.pallas_call(
        _pos_embed_kernel,
        grid=(L // LT,),
        in_specs=[pl.BlockSpec((LT, F), lambda i: (i, 0))],
        out_specs=pl.BlockSpec(memory_space=pl.ANY),
        out_shape=jax.ShapeDtypeStruct((B, F, L), jnp.float32),
        scratch_shapes=[
            pltpu.VMEM((L // LT, F, LT), jnp.float32),
            pltpu.SemaphoreType.DMA((L // LT, B)),
        ],
    )(row_embed)


# speedup vs baseline: 1.2871x; 1.2871x over previous
import jax
import jax.numpy as jnp
from jax.experimental import pallas as pl
from jax.experimental.pallas import tpu as pltpu

_NCH = 4


def _pos_embed_kernel(B, F, L, emb_ref, out_ref, in_v, t_v, in_sems, out_sems):
    LC = L // _NCH

    def in_copy(c):
        return pltpu.make_async_copy(
            emb_ref.at[pl.ds(c * LC, LC), :], in_v.at[c], in_sems.at[c]
        )

    def out_copy(c, b):
        return pltpu.make_async_copy(
            t_v.at[c], out_ref.at[b, :, pl.ds(c * LC, LC)], out_sems.at[c, b]
        )

    for c in range(_NCH):
        in_copy(c).start()
    for c in range(_NCH):
        in_copy(c).wait()
        t_v[c] = in_v[c].T
        for b in range(B):
            out_copy(c, b).start()
    for c in range(_NCH):
        for b in range(B):
            out_copy(c, b).wait()


def kernel(x, mask, row_embed):
    B = x.shape[0]
    F = x.shape[1]
    L = x.shape[-1]
    LC = L // _NCH
    import functools

    return pl.pallas_call(
        functools.partial(_pos_embed_kernel, B, F, L),
        grid=(1,),
        in_specs=[pl.BlockSpec(memory_space=pl.ANY)],
        out_specs=pl.BlockSpec(memory_space=pl.ANY),
        out_shape=jax.ShapeDtypeStruct((B, F, L), jnp.float32),
        scratch_shapes=[
            pltpu.VMEM((_NCH, LC, F), jnp.float32),
            pltpu.VMEM((_NCH, F, LC), jnp.float32),
            pltpu.SemaphoreType.DMA((_NCH,)),
            pltpu.SemaphoreType.DMA((_NCH, B)),
        ],
    )(row_embed)
